# baseline (speedup 1.0000x reference)
import functools

import jax
import jax.numpy as jnp
from jax import lax
from jax.experimental import pallas as pl
from jax.experimental.pallas import tpu as pltpu
from jax.experimental.pallas import tpu_sc as plsc

_NC = 2
_NS = 16
_NW = _NC * _NS
_L = 16


@functools.lru_cache(maxsize=None)
def _make_sc_add(n_batch: int, seq: int, d: int):
    seq_per_w = seq // _NW
    chunk_rows = 16
    n_chunks = seq_per_w // chunk_rows
    nb = n_batch
    assert n_chunks % 2 == 0 and n_chunks >= 4 and nb >= 4 and nb % 2 == 0
    mesh = plsc.VectorSubcoreMesh(
        core_axis_name="c", subcore_axis_name="s",
        num_cores=_NC, num_subcores=_NS,
    )

    @functools.partial(
        pl.kernel,
        out_type=jax.ShapeDtypeStruct((n_batch, seq, d), jnp.float32),
        mesh=mesh,
        scratch_types=(
            [pltpu.VMEM((chunk_rows, d), jnp.float32)] * (nb + 2)
            + [pltpu.SemaphoreType.DMA] * (2 * nb + 2)
        ),
    )
    def sc_add(x_hbm, p_hbm, o_hbm, *scratch):
        xb = list(scratch[:nb])
        pb = [scratch[nb], scratch[nb + 1]]
        sems = scratch[nb + 2:]
        sx = list(sems[:nb])
        so = list(sems[nb:2 * nb])
        sp = [sems[2 * nb], sems[2 * nb + 1]]

        wid = lax.axis_index("s") * _NC + lax.axis_index("c")
        row_base = wid * seq_per_w

        def start_x_in(c, b):
            rows = pl.ds(row_base + c * chunk_rows, chunk_rows)
            pltpu.make_async_copy(
                x_hbm.at[b, rows, :], xb[b], sx[b]).start()

        def start_p_in(c, r):
            rows = pl.ds(row_base + c * chunk_rows, chunk_rows)
            pltpu.make_async_copy(p_hbm.at[rows, :], pb[r], sp[r]).start()

        def wait(sem, buf):
            pltpu.make_async_copy(
                p_hbm.at[pl.ds(0, chunk_rows), :], buf, sem).wait()

        for r in range(2):
            start_p_in(r, r)
        for b in range(2):
            start_x_in(0, b)

        def pair_body(g, carry):
            for r in range(2):
                c = g * 2 + r
                wait(sp[r], pb[r])
                for b in range(nb):
                    if b < 2:
                        @pl.when(c >= 1)
                        def _():
                            wait(so[b + 2], xb[b + 2])
                        start_x_in(c, b + 2)
                    else:
                        wait(so[b - 2], xb[b - 2])

                        @pl.when(c + 1 < n_chunks)
                        def _():
                            start_x_in(c + 1, b - 2)

                    wait(sx[b], xb[b])

                    @plsc.parallel_loop(0, d // _L, unroll=2)
                    def _(i):
                        sl = pl.ds(i * _L, _L)
                        for row in range(chunk_rows):
                            plsc.addupdate(xb[b].at[row, sl], pb[r][row, sl])

                    rows = pl.ds(row_base + c * chunk_rows, chunk_rows)
                    pltpu.make_async_copy(
                        xb[b], o_hbm.at[b, rows, :], so[b]).start()

                @pl.when(c + 2 < n_chunks)
                def _():
                    start_p_in(c + 2, r)
            return carry

        lax.fori_loop(0, n_chunks // 2, pair_body, 0)

        for b in range(2, nb):
            wait(so[b], xb[b])

    return sc_add


def kernel(x, pos_table):
    B, S, D = x.shape
    fn = _make_sc_add(B, S, D)
    return fn(x, pos_table[:S])


# speedup vs baseline: 1.0456x; 1.0456x over previous
import functools

import jax
import jax.numpy as jnp
from jax import lax
from jax.experimental import pallas as pl
from jax.experimental.pallas import tpu as pltpu
from jax.experimental.pallas import tpu_sc as plsc

_NC = 2
_NS = 16
_NW = _NC * _NS
_L = 16


@functools.lru_cache(maxsize=None)
def _make_sc_add(n_batch: int, seq: int, d: int):
    seq_per_w = seq // _NW
    chunk_rows = 8
    n_chunks = seq_per_w // chunk_rows
    assert n_chunks % 2 == 0 and n_chunks >= 4
    mesh = plsc.VectorSubcoreMesh(
        core_axis_name="c", subcore_axis_name="s",
        num_cores=_NC, num_subcores=_NS,
    )
    nb = n_batch

    @functools.partial(
        pl.kernel,
        out_type=jax.ShapeDtypeStruct((n_batch, seq, d), jnp.float32),
        mesh=mesh,
        scratch_types=(
            [pltpu.VMEM((chunk_rows, d), jnp.float32)] * (2 * nb + 2)
            + [pltpu.SemaphoreType.DMA] * (4 * nb + 2)
        ),
    )
    def sc_add(x_hbm, p_hbm, o_hbm, *scratch):
        xb = [[scratch[b * 2 + r] for r in range(2)] for b in range(nb)]
        pb = [scratch[2 * nb], scratch[2 * nb + 1]]
        sems = scratch[2 * nb + 2:]
        sx = [[sems[b * 2 + r] for r in range(2)] for b in range(nb)]
        so = [[sems[2 * nb + b * 2 + r] for r in range(2)] for b in range(nb)]
        sp = [sems[4 * nb], sems[4 * nb + 1]]

        wid = lax.axis_index("s") * _NC + lax.axis_index("c")
        row_base = wid * seq_per_w

        def start_x_in(c, b, r):
            rows = pl.ds(row_base + c * chunk_rows, chunk_rows)
            pltpu.make_async_copy(
                x_hbm.at[b, rows, :], xb[b][r], sx[b][r]).start()

        def start_p_in(c, r):
            rows = pl.ds(row_base + c * chunk_rows, chunk_rows)
            pltpu.make_async_copy(p_hbm.at[rows, :], pb[r], sp[r]).start()

        def wait(sem, buf):
            pltpu.make_async_copy(
                p_hbm.at[pl.ds(0, chunk_rows), :], buf, sem).wait()

        for r in range(2):
            start_p_in(r, r)
        for b in range(nb):
            start_x_in(0, b, 0)

        def pair_body(g, carry):
            for r in range(2):
                c = g * 2 + r
                wait(sp[r], pb[r])
                for b in range(nb):
                    @pl.when(c + 1 < n_chunks)
                    def _():
                        @pl.when(c >= 1)
                        def _():
                            wait(so[b][1 - r], xb[b][1 - r])
                        start_x_in(c + 1, b, 1 - r)

                    wait(sx[b][r], xb[b][r])

                    @plsc.parallel_loop(0, d // _L, unroll=2)
                    def _(i):
                        sl = pl.ds(i * _L, _L)
                        for row in range(chunk_rows):
                            plsc.addupdate(
                                xb[b][r].at[row, sl], pb[r][row, sl])

                    rows = pl.ds(row_base + c * chunk_rows, chunk_rows)
                    pltpu.make_async_copy(
                        xb[b][r], o_hbm.at[b, rows, :], so[b][r]).start()

                @pl.when(c + 2 < n_chunks)
                def _():
                    start_p_in(c + 2, r)
            return carry

        lax.fori_loop(0, n_chunks // 2, pair_body, 0)

        for b in range(nb):
            for r in range(2):
                wait(so[b][r], xb[b][r])

    return sc_add


def kernel(x, pos_table):
    B, S, D = x.shape
    fn = _make_sc_add(B, S, D)
    pos = pos_table if pos_table.shape[0] == S else pos_table[:S]
    return fn(x, pos)
